# baseline (speedup 1.0000x reference)
import functools

import jax
import jax.numpy as jnp
from jax import lax
from jax.experimental import pallas as pl
from jax.experimental.pallas import tpu as pltpu
from jax.experimental.pallas import tpu_sc as plsc

N = 10000
C = 128
E = 320000

NW = 32
NPAD = 10240
CHUNK = 128
NCHUNK = 80
WIN = 16
NWIN = NCHUNK // WIN
E_PER_W = NCHUNK * CHUNK
EPAD = NW * E_PER_W
ROWS_PER_TILE = NPAD // 16

_MESH = plsc.VectorSubcoreMesh(core_axis_name="c", subcore_axis_name="s")


@functools.partial(
    pl.kernel,
    out_type=jax.ShapeDtypeStruct((NW, NPAD), jnp.float32),
    mesh=_MESH,
    scratch_types=[
        pltpu.VMEM((1024,), jnp.int32),
        pltpu.VMEM((NPAD,), jnp.float32),
    ],
    compiler_params=pltpu.CompilerParams(needs_layout_passes=False),
)
def _deg_kernel(dst_hbm, degp_hbm, dst_v, deg_v):
    c = lax.axis_index("c")
    s = lax.axis_index("s")
    wid = s * 2 + c

    zero16 = jnp.zeros((16,), jnp.float32)

    def zbody(i, carry):
        deg_v[pl.ds(i * 16, 16)] = zero16
        return carry

    lax.fori_loop(0, NPAD // 16, zbody, 0)

    ones16 = jnp.ones((16,), jnp.float32)

    def wbody(w, carry):
        pltpu.sync_copy(dst_hbm.at[pl.ds(wid * E_PER_W + w * 1024, 1024)],
                        dst_v)

        def body(i, carry2):
            idx = dst_v[pl.ds(i * 16, 16)]
            plsc.addupdate_scatter(deg_v, [idx], ones16)
            return carry2

        lax.fori_loop(0, 1024 // 16, body, 0)
        return carry

    lax.fori_loop(0, E_PER_W // 1024, wbody, 0)
    pltpu.sync_copy(deg_v, degp_hbm.at[wid])


def _mm_body(x_ref, w_ref, degp_ref, g_ref, dis_ref):
    h = jnp.dot(x_ref[...], w_ref[...], preferred_element_type=jnp.float32)
    deg = jnp.sum(degp_ref[...], axis=0) + 1.0
    dis = lax.rsqrt(deg)
    g_ref[...] = h * dis[:, None]
    dis_ref[...] = dis[:, None]


def _matmul_norm(x_pad, W, degp):
    BM = 256
    return pl.pallas_call(
        _mm_body,
        grid=(NPAD // BM,),
        in_specs=[
            pl.BlockSpec((BM, C), lambda i: (i, 0)),
            pl.BlockSpec((C, C), lambda i: (0, 0)),
            pl.BlockSpec((NW, BM), lambda i: (0, i)),
        ],
        out_specs=[
            pl.BlockSpec((BM, C), lambda i: (i, 0)),
            pl.BlockSpec((BM, 1), lambda i: (i, 0)),
        ],
        out_shape=[
            jax.ShapeDtypeStruct((NPAD, C), jnp.float32),
            jax.ShapeDtypeStruct((NPAD, 1), jnp.float32),
        ],
    )(x_pad, W, degp)


@functools.partial(
    pl.kernel,
    out_type=jax.ShapeDtypeStruct((2, NPAD, C), jnp.float32),
    mesh=_MESH,
    scratch_types=[
        pltpu.VMEM((2, WIN, CHUNK), jnp.int32),
        pltpu.VMEM((2, WIN, CHUNK), jnp.int32),
        pltpu.VMEM((CHUNK, C), jnp.float32),
        pltpu.VMEM((CHUNK, C), jnp.float32),
        pltpu.VMEM_SHARED((NPAD, C), jnp.float32),
        pltpu.SemaphoreType.DMA,
        pltpu.SemaphoreType.DMA,
        pltpu.SemaphoreType.DMA,
        pltpu.SemaphoreType.DMA,
        pltpu.SemaphoreType.DMA,
    ],
    compiler_params=pltpu.CompilerParams(needs_layout_passes=False),
)
def _edge_kernel(src_hbm, dst_hbm, g_hbm, outp_hbm, src_w, dst_w, buf_a,
                 buf_b, S_sh, gsem_a, gsem_b, ssem_a, ssem_b, wsem):
    c = lax.axis_index("c")
    s = lax.axis_index("s")
    wid = s * 2 + c

    bufs = (buf_a, buf_b)
    gsems = (gsem_a, gsem_b)
    ssems = (ssem_a, ssem_b)

    def start_gather(slot, r, p):
        pltpu.async_copy(g_hbm.at[src_w.at[slot, r]], bufs[p], gsems[p])

    def wait_gather(p):
        pltpu.make_async_copy(g_hbm.at[src_w.at[0, 0]], bufs[p],
                              gsems[p]).wait()

    def start_scatter(slot, r, p):
        pltpu.async_copy(bufs[p], S_sh.at[dst_w.at[slot, r]], ssems[p],
                         add=True)

    def wait_scatter(p):
        pltpu.make_async_copy(bufs[p], S_sh.at[dst_w.at[0, 0]],
                              ssems[p]).wait()

    def chunk_step(slot, r, p, gslot, gr):
        wait_gather(p)
        start_scatter(slot, r, p)
        wait_scatter(1 - p)
        start_gather(gslot, gr, 1 - p)

    pltpu.sync_copy(src_hbm.at[wid, pl.ds(0, WIN)], src_w.at[0])
    pltpu.sync_copy(dst_hbm.at[wid, pl.ds(0, WIN)], dst_w.at[0])

    zero16 = jnp.zeros((16,), jnp.float32)

    def zbody(i, carry):
        for j in range(C // 16):
            buf_a[i, pl.ds(j * 16, 16)] = zero16
            buf_b[i, pl.ds(j * 16, 16)] = zero16
        return carry

    lax.fori_loop(0, CHUNK, zbody, 0)

    for k in range(ROWS_PER_TILE // CHUNK):
        pltpu.sync_copy(buf_a, S_sh.at[pl.ds(s * ROWS_PER_TILE + k * CHUNK,
                                             CHUNK)])
    start_gather(0, 0, 0)
    plsc.subcore_barrier()

    start_scatter(0, 0, 1)

    for w in range(NWIN):
        slot = w % 2
        nslot = 1 - slot
        if w + 1 < NWIN:
            pltpu.async_copy(src_hbm.at[wid, pl.ds((w + 1) * WIN, WIN)],
                             src_w.at[nslot], wsem)
            pltpu.async_copy(dst_hbm.at[wid, pl.ds((w + 1) * WIN, WIN)],
                             dst_w.at[nslot], wsem)

        def pbody(i, carry, slot=slot):
            chunk_step(slot, 2 * i, 0, slot, 2 * i + 1)
            chunk_step(slot, 2 * i + 1, 1, slot, 2 * i + 2)
            return carry

        lax.fori_loop(0, (WIN - 2) // 2, pbody, 0)
        chunk_step(slot, WIN - 2, 0, slot, WIN - 1)
        if w + 1 < NWIN:
            pltpu.make_async_copy(src_hbm.at[wid, pl.ds(0, WIN)],
                                  src_w.at[nslot], wsem).wait()
            pltpu.make_async_copy(dst_hbm.at[wid, pl.ds(0, WIN)],
                                  dst_w.at[nslot], wsem).wait()
            chunk_step(slot, WIN - 1, 1, nslot, 0)
        else:
            wait_gather(1)
            start_scatter(slot, WIN - 1, 1)
            wait_scatter(0)
            wait_scatter(1)
    plsc.subcore_barrier()

    def rbody(k, carry):
        base = s * ROWS_PER_TILE + k * CHUNK
        pltpu.sync_copy(S_sh.at[pl.ds(base, CHUNK)], buf_a)
        pltpu.sync_copy(buf_a, outp_hbm.at[c, pl.ds(base, CHUNK)])
        return carry

    lax.fori_loop(0, ROWS_PER_TILE // CHUNK, rbody, 0)


def _fin_body(s0_ref, s1_ref, g_ref, dis_ref, b_ref, o_ref):
    t = (s0_ref[...] + s1_ref[...] + g_ref[...]) * dis_ref[...]
    o_ref[...] = jnp.maximum(t + b_ref[...], 0.0)


def _finish(S0, S1, g, dis, b2):
    BM = 256
    return pl.pallas_call(
        _fin_body,
        grid=(NPAD // BM,),
        in_specs=[
            pl.BlockSpec((BM, C), lambda i: (i, 0)),
            pl.BlockSpec((BM, C), lambda i: (i, 0)),
            pl.BlockSpec((BM, C), lambda i: (i, 0)),
            pl.BlockSpec((BM, 1), lambda i: (i, 0)),
            pl.BlockSpec((1, C), lambda i: (0, 0)),
        ],
        out_specs=pl.BlockSpec((BM, C), lambda i: (i, 0)),
        out_shape=jax.ShapeDtypeStruct((NPAD, C), jnp.float32),
    )(S0, S1, g, dis, b2)


def kernel(x, edge_index, W, b):
    ei = edge_index.astype(jnp.int32)
    pad = jnp.full((EPAD - E,), N, jnp.int32)
    src_p = jnp.concatenate([ei[0], pad])
    dst_p = jnp.concatenate([ei[1], pad])
    src3 = src_p.reshape(NW, NCHUNK, CHUNK)
    dst3 = dst_p.reshape(NW, NCHUNK, CHUNK)

    x_pad = jnp.pad(x, ((0, NPAD - N), (0, 0)))

    degp = _deg_kernel(dst_p)
    g, dis = _matmul_norm(x_pad, W, degp)
    Sp = _edge_kernel(src3, dst3, g)
    out = _finish(Sp[0], Sp[1], g, dis, b.reshape(1, C))
    return out[:N]
